# Initial kernel scaffold; baseline (speedup 1.0000x reference)
import jax
import jax.numpy as jnp
from jax.experimental import pallas as pl


def kernel(x, edge_index, W1, b1, W2, b2):
    raise NotImplementedError("write your pallas kernel here")


# speedup vs baseline: 25.4252x; 25.4252x over previous
import functools

import jax
import jax.numpy as jnp
from jax import lax
from jax.experimental import pallas as pl
from jax.experimental.pallas import tpu as pltpu
from jax.experimental.pallas import tpu_sc as plsc

N = 100000
E = 1600000
NC = 2
NS = 16
NW = NC * NS
NPAD = 100352
RPT = NPAD // NS
C = 128
EPAD = 1638400
EPT = EPAD // NW
NCHUNK = EPT // C

_MESH = plsc.VectorSubcoreMesh(
    core_axis_name="c", subcore_axis_name="s", num_cores=NC, num_subcores=NS
)

_F32 = jnp.float32


def _nslice(ref, s):
    return ref.at[pl.ds(s * RPT, RPT)]


@functools.partial(
    pl.kernel,
    out_type=[
        jax.ShapeDtypeStruct((NPAD,), _F32),
        jax.ShapeDtypeStruct((NPAD,), _F32),
    ],
    mesh=_MESH,
    scratch_types=[
        pltpu.VMEM((C,), jnp.int32),
        pltpu.VMEM((C,), _F32),
        pltpu.VMEM_SHARED((NPAD,), _F32),
    ],
)
def _deg_call(dst_hbm, zeros_hbm, out0_hbm, out1_hbm, didx, ones_v, deg_sh):
    c = lax.axis_index("c")
    s = lax.axis_index("s")
    wid = c * NS + s
    for i in range(C // 16):
        ones_v[pl.ds(i * 16, 16)] = jnp.ones((16,), _F32)
    pltpu.sync_copy(_nslice(zeros_hbm, s), _nslice(deg_sh, s))
    plsc.subcore_barrier()

    ebase = wid * EPT

    def chunk(j, carry):
        pltpu.sync_copy(dst_hbm.at[pl.ds(ebase + j * C, C)], didx)
        pltpu.sync_copy(ones_v, deg_sh.at[didx], add=True)
        return carry

    lax.fori_loop(0, NCHUNK, chunk, 0)
    plsc.subcore_barrier()

    @pl.when(c == 0)
    def _():
        pltpu.sync_copy(_nslice(deg_sh, s), _nslice(out0_hbm, s))

    @pl.when(c != 0)
    def _():
        pltpu.sync_copy(_nslice(deg_sh, s), _nslice(out1_hbm, s))


def _mp_body(F, *refs):
    src_hbm, dst_hbm = refs[0], refs[1]
    g_hbm = refs[2:2 + F]
    z_hbm = refs[2 + F]
    outs = refs[3 + F:3 + 3 * F]
    sidx, didx, m_v = refs[3 + 3 * F:6 + 3 * F]
    g_sh = refs[6 + 3 * F:6 + 4 * F]
    a_sh = refs[6 + 4 * F:6 + 5 * F]

    c = lax.axis_index("c")
    s = lax.axis_index("s")
    wid = c * NS + s

    for f in range(F):
        pltpu.sync_copy(_nslice(g_hbm[f], s), _nslice(g_sh[f], s))

    @pl.when(c == 0)
    def _():
        for f in range(F):
            pltpu.sync_copy(_nslice(g_hbm[f], s), _nslice(a_sh[f], s))

    @pl.when(c != 0)
    def _():
        for f in range(F):
            pltpu.sync_copy(_nslice(z_hbm, s), _nslice(a_sh[f], s))

    plsc.subcore_barrier()

    ebase = wid * EPT

    def chunk(j, carry):
        pltpu.sync_copy(src_hbm.at[pl.ds(ebase + j * C, C)], sidx)
        pltpu.sync_copy(dst_hbm.at[pl.ds(ebase + j * C, C)], didx)
        for f in range(F):
            pltpu.sync_copy(g_sh[f].at[sidx], m_v)
            pltpu.sync_copy(m_v, a_sh[f].at[didx], add=True)
        return carry

    lax.fori_loop(0, NCHUNK, chunk, 0)
    plsc.subcore_barrier()

    @pl.when(c == 0)
    def _():
        for f in range(F):
            pltpu.sync_copy(_nslice(a_sh[f], s), _nslice(outs[f], s))

    @pl.when(c != 0)
    def _():
        for f in range(F):
            pltpu.sync_copy(_nslice(a_sh[f], s), _nslice(outs[F + f], s))


def _make_mp(F):
    return functools.partial(
        pl.kernel,
        out_type=[jax.ShapeDtypeStruct((NPAD,), _F32)] * (2 * F),
        mesh=_MESH,
        scratch_types=(
            [
                pltpu.VMEM((C,), jnp.int32),
                pltpu.VMEM((C,), jnp.int32),
                pltpu.VMEM((C,), _F32),
            ]
            + [pltpu.VMEM_SHARED((NPAD,), _F32)] * (2 * F)
        ),
    )(functools.partial(_mp_body, F))


_mp4_call = _make_mp(4)
_mp2_call = _make_mp(2)


_V2D = (NPAD // 128, 128)


def _vspec():
    return pl.BlockSpec(_V2D, lambda: (0, 0))


def _sspec():
    return pl.BlockSpec(memory_space=pltpu.SMEM)


def _mm1t_body(w1t_ref, xt_ref, h_ref):
    h_ref[...] = jnp.dot(w1t_ref[...], xt_ref[...], preferred_element_type=_F32)


def _tc_mm1t(w1t, xpt):
    return pl.pallas_call(
        _mm1t_body,
        grid=(NS,),
        in_specs=[
            pl.BlockSpec((4, 34), lambda i: (0, 0)),
            pl.BlockSpec((34, RPT), lambda i: (0, i)),
        ],
        out_specs=pl.BlockSpec((4, RPT), lambda i: (0, i)),
        out_shape=jax.ShapeDtypeStruct((4, NPAD), _F32),
    )(w1t, xpt)


def _disg1_body(d0_ref, d1_ref, h0_ref, h1_ref, h2_ref, h3_ref,
                dis_ref, g0_ref, g1_ref, g2_ref, g3_ref):
    dis = lax.rsqrt(d0_ref[...] + d1_ref[...] + 1.0)
    dis_ref[...] = dis
    g0_ref[...] = h0_ref[...] * dis
    g1_ref[...] = h1_ref[...] * dis
    g2_ref[...] = h2_ref[...] * dis
    g3_ref[...] = h3_ref[...] * dis


def _tc_disg1(d0, d1, h):
    return pl.pallas_call(
        _disg1_body,
        in_specs=[_vspec()] * 6,
        out_specs=[_vspec()] * 5,
        out_shape=[jax.ShapeDtypeStruct(_V2D, _F32)] * 5,
    )(d0.reshape(_V2D), d1.reshape(_V2D),
      h[0].reshape(_V2D), h[1].reshape(_V2D),
      h[2].reshape(_V2D), h[3].reshape(_V2D))


def _mid_body(dis_ref, a00, a01, a02, a03, a10, a11, a12, a13,
              b1_ref, w2_ref, g0_ref, g1_ref):
    dis = dis_ref[...]
    o = [
        jnp.maximum(dis * (a0[...] + a1[...]) + b1_ref[f], 0.0)
        for f, (a0, a1) in enumerate(
            [(a00, a10), (a01, a11), (a02, a12), (a03, a13)]
        )
    ]
    for j, g_ref in enumerate((g0_ref, g1_ref)):
        h = o[0] * w2_ref[0 * 2 + j]
        for f in range(1, 4):
            h = h + o[f] * w2_ref[f * 2 + j]
        g_ref[...] = h * dis


def _tc_mid(dis, a, b1, w2f):
    return pl.pallas_call(
        _mid_body,
        in_specs=[_vspec()] * 9 + [_sspec(), _sspec()],
        out_specs=[_vspec()] * 2,
        out_shape=[jax.ShapeDtypeStruct(_V2D, _F32)] * 2,
    )(dis, *[x.reshape(_V2D) for x in a], b1, w2f)


def _fin_body(dis_ref, a00, a01, a10, a11, b2_ref, o0_ref, o1_ref):
    dis = dis_ref[...]
    z0 = dis * (a00[...] + a10[...]) + b2_ref[0]
    z1 = dis * (a01[...] + a11[...]) + b2_ref[1]
    m = jnp.maximum(z0, z1)
    e0 = jnp.exp(z0 - m)
    e1 = jnp.exp(z1 - m)
    inv = 1.0 / (e0 + e1)
    o0_ref[...] = e0 * inv
    o1_ref[...] = e1 * inv


def _tc_fin(dis, a, b2):
    return pl.pallas_call(
        _fin_body,
        in_specs=[_vspec()] * 5 + [_sspec()],
        out_specs=[_vspec()] * 2,
        out_shape=[jax.ShapeDtypeStruct(_V2D, _F32)] * 2,
    )(dis, *[x.reshape(_V2D) for x in a], b2)


def kernel(x, edge_index, W1, b1, W2, b2):
    xpt = jnp.zeros((NPAD, 34), _F32).at[:N].set(x).T
    npe = EPAD - E
    padv = (N + (jnp.arange(npe, dtype=jnp.int32) % (NPAD - N))).astype(jnp.int32)
    srcp = jnp.concatenate([edge_index[0], padv])
    dstp = jnp.concatenate([edge_index[1], padv])
    zd = jnp.zeros((NPAD,), _F32)

    d0, d1 = _deg_call(dstp, zd)
    h1t = _tc_mm1t(W1.T, xpt)
    dis2d, g0, g1, g2, g3 = _tc_disg1(d0, d1, [h1t[f] for f in range(4)])
    a1 = _mp4_call(srcp, dstp, g0.reshape(NPAD), g1.reshape(NPAD),
                   g2.reshape(NPAD), g3.reshape(NPAD), zd)
    g20, g21 = _tc_mid(dis2d, a1, b1, W2.reshape(8))
    a2 = _mp2_call(srcp, dstp, g20.reshape(NPAD), g21.reshape(NPAD), zd)
    o0, o1 = _tc_fin(dis2d, a2, b2)
    return jnp.stack([o0.reshape(NPAD)[:N], o1.reshape(NPAD)[:N]], axis=1)


# speedup vs baseline: 58.7271x; 2.3098x over previous
import functools

import jax
import jax.numpy as jnp
from jax import lax
from jax.experimental import pallas as pl
from jax.experimental.pallas import tpu as pltpu
from jax.experimental.pallas import tpu_sc as plsc

N = 100000
E = 1600000
NC = 2
NS = 16
NW = NC * NS
NPAD = 100352
RPT = NPAD // NS
C = 512
EPAD = 1638400
EPT = EPAD // NW
NCHUNK = EPT // C

_MESH = plsc.VectorSubcoreMesh(
    core_axis_name="c", subcore_axis_name="s", num_cores=NC, num_subcores=NS
)

_F32 = jnp.float32


def _nslice(ref, s):
    return ref.at[pl.ds(s * RPT, RPT)]


@functools.partial(
    pl.kernel,
    out_type=[
        jax.ShapeDtypeStruct((NPAD,), _F32),
        jax.ShapeDtypeStruct((NPAD,), _F32),
    ],
    mesh=_MESH,
    scratch_types=[
        pltpu.VMEM((C,), jnp.int32),
        pltpu.VMEM((C,), _F32),
        pltpu.VMEM_SHARED((NPAD,), _F32),
    ],
)
def _deg_call(dst_hbm, zeros_hbm, out0_hbm, out1_hbm, didx, ones_v, deg_sh):
    c = lax.axis_index("c")
    s = lax.axis_index("s")
    wid = c * NS + s
    for i in range(C // 16):
        ones_v[pl.ds(i * 16, 16)] = jnp.ones((16,), _F32)
    pltpu.sync_copy(_nslice(zeros_hbm, s), _nslice(deg_sh, s))
    plsc.subcore_barrier()

    ebase = wid * EPT

    def chunk(j, carry):
        pltpu.sync_copy(dst_hbm.at[pl.ds(ebase + j * C, C)], didx)
        pltpu.sync_copy(ones_v, deg_sh.at[didx], add=True)
        return carry

    lax.fori_loop(0, NCHUNK, chunk, 0)
    plsc.subcore_barrier()

    @pl.when(c == 0)
    def _():
        pltpu.sync_copy(_nslice(deg_sh, s), _nslice(out0_hbm, s))

    @pl.when(c != 0)
    def _():
        pltpu.sync_copy(_nslice(deg_sh, s), _nslice(out1_hbm, s))


def _mp_body(F, *refs):
    src_hbm, dst_hbm = refs[0], refs[1]
    g_hbm = refs[2:2 + F]
    z_hbm = refs[2 + F]
    outs = refs[3 + F:3 + 3 * F]
    sidx, didx, m_v = refs[3 + 3 * F:6 + 3 * F]
    g_sh = refs[6 + 3 * F:6 + 4 * F]
    a_sh = refs[6 + 4 * F:6 + 5 * F]

    c = lax.axis_index("c")
    s = lax.axis_index("s")
    wid = c * NS + s

    for f in range(F):
        pltpu.sync_copy(_nslice(g_hbm[f], s), _nslice(g_sh[f], s))

    @pl.when(c == 0)
    def _():
        for f in range(F):
            pltpu.sync_copy(_nslice(g_hbm[f], s), _nslice(a_sh[f], s))

    @pl.when(c != 0)
    def _():
        for f in range(F):
            pltpu.sync_copy(_nslice(z_hbm, s), _nslice(a_sh[f], s))

    plsc.subcore_barrier()

    ebase = wid * EPT

    def chunk(j, carry):
        pltpu.sync_copy(src_hbm.at[pl.ds(ebase + j * C, C)], sidx)
        pltpu.sync_copy(dst_hbm.at[pl.ds(ebase + j * C, C)], didx)
        for f in range(F):
            pltpu.sync_copy(g_sh[f].at[sidx], m_v)
            pltpu.sync_copy(m_v, a_sh[f].at[didx], add=True)
        return carry

    lax.fori_loop(0, NCHUNK, chunk, 0)
    plsc.subcore_barrier()

    @pl.when(c == 0)
    def _():
        for f in range(F):
            pltpu.sync_copy(_nslice(a_sh[f], s), _nslice(outs[f], s))

    @pl.when(c != 0)
    def _():
        for f in range(F):
            pltpu.sync_copy(_nslice(a_sh[f], s), _nslice(outs[F + f], s))


def _make_mp(F):
    return functools.partial(
        pl.kernel,
        out_type=[jax.ShapeDtypeStruct((NPAD,), _F32)] * (2 * F),
        mesh=_MESH,
        scratch_types=(
            [
                pltpu.VMEM((C,), jnp.int32),
                pltpu.VMEM((C,), jnp.int32),
                pltpu.VMEM((C,), _F32),
            ]
            + [pltpu.VMEM_SHARED((NPAD,), _F32)] * (2 * F)
        ),
    )(functools.partial(_mp_body, F))


_mp4_call = _make_mp(4)
_mp2_call = _make_mp(2)


_V2D = (NPAD // 128, 128)


def _vspec():
    return pl.BlockSpec(_V2D, lambda: (0, 0))


def _sspec():
    return pl.BlockSpec(memory_space=pltpu.SMEM)


def _mm1t_body(w1t_ref, xt_ref, h_ref):
    h_ref[...] = jnp.dot(w1t_ref[...], xt_ref[...], preferred_element_type=_F32)


def _tc_mm1t(w1t, xpt):
    return pl.pallas_call(
        _mm1t_body,
        grid=(NS,),
        in_specs=[
            pl.BlockSpec((4, 34), lambda i: (0, 0)),
            pl.BlockSpec((34, RPT), lambda i: (0, i)),
        ],
        out_specs=pl.BlockSpec((4, RPT), lambda i: (0, i)),
        out_shape=jax.ShapeDtypeStruct((4, NPAD), _F32),
    )(w1t, xpt)


def _disg1_body(d0_ref, d1_ref, h0_ref, h1_ref, h2_ref, h3_ref,
                dis_ref, g0_ref, g1_ref, g2_ref, g3_ref):
    dis = lax.rsqrt(d0_ref[...] + d1_ref[...] + 1.0)
    dis_ref[...] = dis
    g0_ref[...] = h0_ref[...] * dis
    g1_ref[...] = h1_ref[...] * dis
    g2_ref[...] = h2_ref[...] * dis
    g3_ref[...] = h3_ref[...] * dis


def _tc_disg1(d0, d1, h):
    return pl.pallas_call(
        _disg1_body,
        in_specs=[_vspec()] * 6,
        out_specs=[_vspec()] * 5,
        out_shape=[jax.ShapeDtypeStruct(_V2D, _F32)] * 5,
    )(d0.reshape(_V2D), d1.reshape(_V2D),
      h[0].reshape(_V2D), h[1].reshape(_V2D),
      h[2].reshape(_V2D), h[3].reshape(_V2D))


def _mid_body(dis_ref, a00, a01, a02, a03, a10, a11, a12, a13,
              b1_ref, w2_ref, g0_ref, g1_ref):
    dis = dis_ref[...]
    o = [
        jnp.maximum(dis * (a0[...] + a1[...]) + b1_ref[f], 0.0)
        for f, (a0, a1) in enumerate(
            [(a00, a10), (a01, a11), (a02, a12), (a03, a13)]
        )
    ]
    for j, g_ref in enumerate((g0_ref, g1_ref)):
        h = o[0] * w2_ref[0 * 2 + j]
        for f in range(1, 4):
            h = h + o[f] * w2_ref[f * 2 + j]
        g_ref[...] = h * dis


def _tc_mid(dis, a, b1, w2f):
    return pl.pallas_call(
        _mid_body,
        in_specs=[_vspec()] * 9 + [_sspec(), _sspec()],
        out_specs=[_vspec()] * 2,
        out_shape=[jax.ShapeDtypeStruct(_V2D, _F32)] * 2,
    )(dis, *[x.reshape(_V2D) for x in a], b1, w2f)


def _fin_body(dis_ref, a00, a01, a10, a11, b2_ref, o0_ref, o1_ref):
    dis = dis_ref[...]
    z0 = dis * (a00[...] + a10[...]) + b2_ref[0]
    z1 = dis * (a01[...] + a11[...]) + b2_ref[1]
    m = jnp.maximum(z0, z1)
    e0 = jnp.exp(z0 - m)
    e1 = jnp.exp(z1 - m)
    inv = 1.0 / (e0 + e1)
    o0_ref[...] = e0 * inv
    o1_ref[...] = e1 * inv


def _tc_fin(dis, a, b2):
    return pl.pallas_call(
        _fin_body,
        in_specs=[_vspec()] * 5 + [_sspec()],
        out_specs=[_vspec()] * 2,
        out_shape=[jax.ShapeDtypeStruct(_V2D, _F32)] * 2,
    )(dis, *[x.reshape(_V2D) for x in a], b2)


def kernel(x, edge_index, W1, b1, W2, b2):
    xpt = jnp.zeros((NPAD, 34), _F32).at[:N].set(x).T
    npe = EPAD - E
    padv = (N + (jnp.arange(npe, dtype=jnp.int32) % (NPAD - N))).astype(jnp.int32)
    srcp = jnp.concatenate([edge_index[0], padv])
    dstp = jnp.concatenate([edge_index[1], padv])
    zd = jnp.zeros((NPAD,), _F32)

    d0, d1 = _deg_call(dstp, zd)
    h1t = _tc_mm1t(W1.T, xpt)
    dis2d, g0, g1, g2, g3 = _tc_disg1(d0, d1, [h1t[f] for f in range(4)])
    a1 = _mp4_call(srcp, dstp, g0.reshape(NPAD), g1.reshape(NPAD),
                   g2.reshape(NPAD), g3.reshape(NPAD), zd)
    g20, g21 = _tc_mid(dis2d, a1, b1, W2.reshape(8))
    a2 = _mp2_call(srcp, dstp, g20.reshape(NPAD), g21.reshape(NPAD), zd)
    o0, o1 = _tc_fin(dis2d, a2, b2)
    return jnp.stack([o0.reshape(NPAD)[:N], o1.reshape(NPAD)[:N]], axis=1)


# speedup vs baseline: 76.7910x; 1.3076x over previous
import functools

import jax
import jax.numpy as jnp
from jax import lax
from jax.experimental import pallas as pl
from jax.experimental.pallas import tpu as pltpu
from jax.experimental.pallas import tpu_sc as plsc

N = 100000
E = 1600000
NC = 2
NS = 16
NW = NC * NS
NPAD = 100352
RPT = NPAD // NS
C = 2048
EPAD = 1638400
EPT = EPAD // NW
NCHUNK = EPT // C

_MESH = plsc.VectorSubcoreMesh(
    core_axis_name="c", subcore_axis_name="s", num_cores=NC, num_subcores=NS
)

_F32 = jnp.float32


def _nslice(ref, s):
    return ref.at[pl.ds(s * RPT, RPT)]


@functools.partial(
    pl.kernel,
    out_type=[
        jax.ShapeDtypeStruct((NPAD,), _F32),
        jax.ShapeDtypeStruct((NPAD,), _F32),
    ],
    mesh=_MESH,
    scratch_types=[
        pltpu.VMEM((C,), jnp.int32),
        pltpu.VMEM((C,), _F32),
        pltpu.VMEM_SHARED((NPAD,), _F32),
    ],
)
def _deg_call(dst_hbm, zeros_hbm, out0_hbm, out1_hbm, didx, ones_v, deg_sh):
    c = lax.axis_index("c")
    s = lax.axis_index("s")
    wid = c * NS + s
    for i in range(C // 16):
        ones_v[pl.ds(i * 16, 16)] = jnp.ones((16,), _F32)
    pltpu.sync_copy(_nslice(zeros_hbm, s), _nslice(deg_sh, s))
    plsc.subcore_barrier()

    ebase = wid * EPT

    def chunk(j, carry):
        pltpu.sync_copy(dst_hbm.at[pl.ds(ebase + j * C, C)], didx)
        pltpu.sync_copy(ones_v, deg_sh.at[didx], add=True)
        return carry

    lax.fori_loop(0, NCHUNK, chunk, 0)
    plsc.subcore_barrier()

    @pl.when(c == 0)
    def _():
        pltpu.sync_copy(_nslice(deg_sh, s), _nslice(out0_hbm, s))

    @pl.when(c != 0)
    def _():
        pltpu.sync_copy(_nslice(deg_sh, s), _nslice(out1_hbm, s))


def _mp_body(F, *refs):
    src_hbm, dst_hbm = refs[0], refs[1]
    g_hbm = refs[2:2 + F]
    z_hbm = refs[2 + F]
    outs = refs[3 + F:3 + 3 * F]
    sidx, didx, m_v = refs[3 + 3 * F:6 + 3 * F]
    g_sh = refs[6 + 3 * F:6 + 4 * F]
    a_sh = refs[6 + 4 * F:6 + 5 * F]

    c = lax.axis_index("c")
    s = lax.axis_index("s")
    wid = c * NS + s

    for f in range(F):
        pltpu.sync_copy(_nslice(g_hbm[f], s), _nslice(g_sh[f], s))

    @pl.when(c == 0)
    def _():
        for f in range(F):
            pltpu.sync_copy(_nslice(g_hbm[f], s), _nslice(a_sh[f], s))

    @pl.when(c != 0)
    def _():
        for f in range(F):
            pltpu.sync_copy(_nslice(z_hbm, s), _nslice(a_sh[f], s))

    plsc.subcore_barrier()

    ebase = wid * EPT

    def chunk(j, carry):
        pltpu.sync_copy(src_hbm.at[pl.ds(ebase + j * C, C)], sidx)
        pltpu.sync_copy(dst_hbm.at[pl.ds(ebase + j * C, C)], didx)
        for f in range(F):
            pltpu.sync_copy(g_sh[f].at[sidx], m_v)
            pltpu.sync_copy(m_v, a_sh[f].at[didx], add=True)
        return carry

    lax.fori_loop(0, NCHUNK, chunk, 0)
    plsc.subcore_barrier()

    @pl.when(c == 0)
    def _():
        for f in range(F):
            pltpu.sync_copy(_nslice(a_sh[f], s), _nslice(outs[f], s))

    @pl.when(c != 0)
    def _():
        for f in range(F):
            pltpu.sync_copy(_nslice(a_sh[f], s), _nslice(outs[F + f], s))


def _make_mp(F):
    return functools.partial(
        pl.kernel,
        out_type=[jax.ShapeDtypeStruct((NPAD,), _F32)] * (2 * F),
        mesh=_MESH,
        scratch_types=(
            [
                pltpu.VMEM((C,), jnp.int32),
                pltpu.VMEM((C,), jnp.int32),
                pltpu.VMEM((C,), _F32),
            ]
            + [pltpu.VMEM_SHARED((NPAD,), _F32)] * (2 * F)
        ),
    )(functools.partial(_mp_body, F))


_mp4_call = _make_mp(4)
_mp2_call = _make_mp(2)


_V2D = (NPAD // 128, 128)


def _vspec():
    return pl.BlockSpec(_V2D, lambda: (0, 0))


def _sspec():
    return pl.BlockSpec(memory_space=pltpu.SMEM)


def _mm1t_body(w1t_ref, xt_ref, h_ref):
    h_ref[...] = jnp.dot(w1t_ref[...], xt_ref[...], preferred_element_type=_F32)


def _tc_mm1t(w1t, xpt):
    return pl.pallas_call(
        _mm1t_body,
        grid=(NS,),
        in_specs=[
            pl.BlockSpec((4, 34), lambda i: (0, 0)),
            pl.BlockSpec((34, RPT), lambda i: (0, i)),
        ],
        out_specs=pl.BlockSpec((4, RPT), lambda i: (0, i)),
        out_shape=jax.ShapeDtypeStruct((4, NPAD), _F32),
    )(w1t, xpt)


def _disg1_body(d0_ref, d1_ref, h0_ref, h1_ref, h2_ref, h3_ref,
                dis_ref, g0_ref, g1_ref, g2_ref, g3_ref):
    dis = lax.rsqrt(d0_ref[...] + d1_ref[...] + 1.0)
    dis_ref[...] = dis
    g0_ref[...] = h0_ref[...] * dis
    g1_ref[...] = h1_ref[...] * dis
    g2_ref[...] = h2_ref[...] * dis
    g3_ref[...] = h3_ref[...] * dis


def _tc_disg1(d0, d1, h):
    return pl.pallas_call(
        _disg1_body,
        in_specs=[_vspec()] * 6,
        out_specs=[_vspec()] * 5,
        out_shape=[jax.ShapeDtypeStruct(_V2D, _F32)] * 5,
    )(d0.reshape(_V2D), d1.reshape(_V2D),
      h[0].reshape(_V2D), h[1].reshape(_V2D),
      h[2].reshape(_V2D), h[3].reshape(_V2D))


def _mid_body(dis_ref, a00, a01, a02, a03, a10, a11, a12, a13,
              b1_ref, w2_ref, g0_ref, g1_ref):
    dis = dis_ref[...]
    o = [
        jnp.maximum(dis * (a0[...] + a1[...]) + b1_ref[f], 0.0)
        for f, (a0, a1) in enumerate(
            [(a00, a10), (a01, a11), (a02, a12), (a03, a13)]
        )
    ]
    for j, g_ref in enumerate((g0_ref, g1_ref)):
        h = o[0] * w2_ref[0 * 2 + j]
        for f in range(1, 4):
            h = h + o[f] * w2_ref[f * 2 + j]
        g_ref[...] = h * dis


def _tc_mid(dis, a, b1, w2f):
    return pl.pallas_call(
        _mid_body,
        in_specs=[_vspec()] * 9 + [_sspec(), _sspec()],
        out_specs=[_vspec()] * 2,
        out_shape=[jax.ShapeDtypeStruct(_V2D, _F32)] * 2,
    )(dis, *[x.reshape(_V2D) for x in a], b1, w2f)


def _fin_body(dis_ref, a00, a01, a10, a11, b2_ref, o0_ref, o1_ref):
    dis = dis_ref[...]
    z0 = dis * (a00[...] + a10[...]) + b2_ref[0]
    z1 = dis * (a01[...] + a11[...]) + b2_ref[1]
    m = jnp.maximum(z0, z1)
    e0 = jnp.exp(z0 - m)
    e1 = jnp.exp(z1 - m)
    inv = 1.0 / (e0 + e1)
    o0_ref[...] = e0 * inv
    o1_ref[...] = e1 * inv


def _tc_fin(dis, a, b2):
    return pl.pallas_call(
        _fin_body,
        in_specs=[_vspec()] * 5 + [_sspec()],
        out_specs=[_vspec()] * 2,
        out_shape=[jax.ShapeDtypeStruct(_V2D, _F32)] * 2,
    )(dis, *[x.reshape(_V2D) for x in a], b2)


def kernel(x, edge_index, W1, b1, W2, b2):
    xpt = jnp.zeros((NPAD, 34), _F32).at[:N].set(x).T
    npe = EPAD - E
    padv = (N + (jnp.arange(npe, dtype=jnp.int32) % (NPAD - N))).astype(jnp.int32)
    srcp = jnp.concatenate([edge_index[0], padv])
    dstp = jnp.concatenate([edge_index[1], padv])
    zd = jnp.zeros((NPAD,), _F32)

    d0, d1 = _deg_call(dstp, zd)
    h1t = _tc_mm1t(W1.T, xpt)
    dis2d, g0, g1, g2, g3 = _tc_disg1(d0, d1, [h1t[f] for f in range(4)])
    a1 = _mp4_call(srcp, dstp, g0.reshape(NPAD), g1.reshape(NPAD),
                   g2.reshape(NPAD), g3.reshape(NPAD), zd)
    g20, g21 = _tc_mid(dis2d, a1, b1, W2.reshape(8))
    a2 = _mp2_call(srcp, dstp, g20.reshape(NPAD), g21.reshape(NPAD), zd)
    o0, o1 = _tc_fin(dis2d, a2, b2)
    return jnp.stack([o0.reshape(NPAD)[:N], o1.reshape(NPAD)[:N]], axis=1)


# speedup vs baseline: 117.5545x; 1.5308x over previous
import functools

import jax
import jax.numpy as jnp
from jax import lax
from jax.experimental import pallas as pl
from jax.experimental.pallas import tpu as pltpu
from jax.experimental.pallas import tpu_sc as plsc

N = 100000
E = 1600000
NC = 2
NS = 16
NW = NC * NS
NPAD = 100352
RPT = NPAD // NS
C = 12800
EPAD = 1638400
EPT = EPAD // NW
NCHUNK = EPT // C

_MESH = plsc.VectorSubcoreMesh(
    core_axis_name="c", subcore_axis_name="s", num_cores=NC, num_subcores=NS
)

_F32 = jnp.float32


def _nslice(ref, s):
    return ref.at[pl.ds(s * RPT, RPT)]


@functools.partial(
    pl.kernel,
    out_type=[
        jax.ShapeDtypeStruct((NPAD,), _F32),
        jax.ShapeDtypeStruct((NPAD,), _F32),
    ],
    mesh=_MESH,
    scratch_types=[
        pltpu.VMEM((C,), jnp.int32),
        pltpu.VMEM((C,), _F32),
        pltpu.VMEM_SHARED((NPAD,), _F32),
    ],
)
def _deg_call(dst_hbm, zeros_hbm, out0_hbm, out1_hbm, didx, ones_v, deg_sh):
    c = lax.axis_index("c")
    s = lax.axis_index("s")
    wid = c * NS + s
    for i in range(C // 16):
        ones_v[pl.ds(i * 16, 16)] = jnp.ones((16,), _F32)
    pltpu.sync_copy(_nslice(zeros_hbm, s), _nslice(deg_sh, s))
    plsc.subcore_barrier()

    ebase = wid * EPT

    def chunk(j, carry):
        pltpu.sync_copy(dst_hbm.at[pl.ds(ebase + j * C, C)], didx)
        pltpu.sync_copy(ones_v, deg_sh.at[didx], add=True)
        return carry

    lax.fori_loop(0, NCHUNK, chunk, 0)
    plsc.subcore_barrier()

    @pl.when(c == 0)
    def _():
        pltpu.sync_copy(_nslice(deg_sh, s), _nslice(out0_hbm, s))

    @pl.when(c != 0)
    def _():
        pltpu.sync_copy(_nslice(deg_sh, s), _nslice(out1_hbm, s))


def _mp_body(F, *refs):
    src_hbm, dst_hbm = refs[0], refs[1]
    g_hbm = refs[2:2 + F]
    z_hbm = refs[2 + F]
    outs = refs[3 + F:3 + 3 * F]
    sidx, didx, m_v = refs[3 + 3 * F:6 + 3 * F]
    g_sh = refs[6 + 3 * F:6 + 4 * F]
    a_sh = refs[6 + 4 * F:6 + 5 * F]

    c = lax.axis_index("c")
    s = lax.axis_index("s")
    wid = c * NS + s

    for f in range(F):
        pltpu.sync_copy(_nslice(g_hbm[f], s), _nslice(g_sh[f], s))

    @pl.when(c == 0)
    def _():
        for f in range(F):
            pltpu.sync_copy(_nslice(g_hbm[f], s), _nslice(a_sh[f], s))

    @pl.when(c != 0)
    def _():
        for f in range(F):
            pltpu.sync_copy(_nslice(z_hbm, s), _nslice(a_sh[f], s))

    plsc.subcore_barrier()

    ebase = wid * EPT

    def chunk(j, carry):
        pltpu.sync_copy(src_hbm.at[pl.ds(ebase + j * C, C)], sidx)
        pltpu.sync_copy(dst_hbm.at[pl.ds(ebase + j * C, C)], didx)
        for f in range(F):
            pltpu.sync_copy(g_sh[f].at[sidx], m_v)
            pltpu.sync_copy(m_v, a_sh[f].at[didx], add=True)
        return carry

    lax.fori_loop(0, NCHUNK, chunk, 0)
    plsc.subcore_barrier()

    @pl.when(c == 0)
    def _():
        for f in range(F):
            pltpu.sync_copy(_nslice(a_sh[f], s), _nslice(outs[f], s))

    @pl.when(c != 0)
    def _():
        for f in range(F):
            pltpu.sync_copy(_nslice(a_sh[f], s), _nslice(outs[F + f], s))


def _make_mp(F):
    return functools.partial(
        pl.kernel,
        out_type=[jax.ShapeDtypeStruct((NPAD,), _F32)] * (2 * F),
        mesh=_MESH,
        scratch_types=(
            [
                pltpu.VMEM((C,), jnp.int32),
                pltpu.VMEM((C,), jnp.int32),
                pltpu.VMEM((C,), _F32),
            ]
            + [pltpu.VMEM_SHARED((NPAD,), _F32)] * (2 * F)
        ),
    )(functools.partial(_mp_body, F))


_mp4_call = _make_mp(4)
_mp2_call = _make_mp(2)


_V2D = (NPAD // 128, 128)


def _vspec():
    return pl.BlockSpec(_V2D, lambda: (0, 0))


def _sspec():
    return pl.BlockSpec(memory_space=pltpu.SMEM)


def _mm1t_body(w1t_ref, xt_ref, h_ref):
    h_ref[...] = jnp.dot(w1t_ref[...], xt_ref[...], preferred_element_type=_F32)


def _tc_mm1t(w1t, xpt):
    return pl.pallas_call(
        _mm1t_body,
        grid=(NS,),
        in_specs=[
            pl.BlockSpec((4, 34), lambda i: (0, 0)),
            pl.BlockSpec((34, RPT), lambda i: (0, i)),
        ],
        out_specs=pl.BlockSpec((4, RPT), lambda i: (0, i)),
        out_shape=jax.ShapeDtypeStruct((4, NPAD), _F32),
    )(w1t, xpt)


def _disg1_body(d0_ref, d1_ref, h0_ref, h1_ref, h2_ref, h3_ref,
                dis_ref, g0_ref, g1_ref, g2_ref, g3_ref):
    dis = lax.rsqrt(d0_ref[...] + d1_ref[...] + 1.0)
    dis_ref[...] = dis
    g0_ref[...] = h0_ref[...] * dis
    g1_ref[...] = h1_ref[...] * dis
    g2_ref[...] = h2_ref[...] * dis
    g3_ref[...] = h3_ref[...] * dis


def _tc_disg1(d0, d1, h):
    return pl.pallas_call(
        _disg1_body,
        in_specs=[_vspec()] * 6,
        out_specs=[_vspec()] * 5,
        out_shape=[jax.ShapeDtypeStruct(_V2D, _F32)] * 5,
    )(d0.reshape(_V2D), d1.reshape(_V2D),
      h[0].reshape(_V2D), h[1].reshape(_V2D),
      h[2].reshape(_V2D), h[3].reshape(_V2D))


def _mid_body(dis_ref, a00, a01, a02, a03, a10, a11, a12, a13,
              b1_ref, w2_ref, g0_ref, g1_ref):
    dis = dis_ref[...]
    o = [
        jnp.maximum(dis * (a0[...] + a1[...]) + b1_ref[f], 0.0)
        for f, (a0, a1) in enumerate(
            [(a00, a10), (a01, a11), (a02, a12), (a03, a13)]
        )
    ]
    for j, g_ref in enumerate((g0_ref, g1_ref)):
        h = o[0] * w2_ref[0 * 2 + j]
        for f in range(1, 4):
            h = h + o[f] * w2_ref[f * 2 + j]
        g_ref[...] = h * dis


def _tc_mid(dis, a, b1, w2f):
    return pl.pallas_call(
        _mid_body,
        in_specs=[_vspec()] * 9 + [_sspec(), _sspec()],
        out_specs=[_vspec()] * 2,
        out_shape=[jax.ShapeDtypeStruct(_V2D, _F32)] * 2,
    )(dis, *[x.reshape(_V2D) for x in a], b1, w2f)


def _fin_body(dis_ref, a00, a01, a10, a11, b2_ref, o0_ref, o1_ref):
    dis = dis_ref[...]
    z0 = dis * (a00[...] + a10[...]) + b2_ref[0]
    z1 = dis * (a01[...] + a11[...]) + b2_ref[1]
    m = jnp.maximum(z0, z1)
    e0 = jnp.exp(z0 - m)
    e1 = jnp.exp(z1 - m)
    inv = 1.0 / (e0 + e1)
    o0_ref[...] = e0 * inv
    o1_ref[...] = e1 * inv


def _tc_fin(dis, a, b2):
    return pl.pallas_call(
        _fin_body,
        in_specs=[_vspec()] * 5 + [_sspec()],
        out_specs=[_vspec()] * 2,
        out_shape=[jax.ShapeDtypeStruct(_V2D, _F32)] * 2,
    )(dis, *[x.reshape(_V2D) for x in a], b2)


def kernel(x, edge_index, W1, b1, W2, b2):
    xpt = jnp.zeros((NPAD, 34), _F32).at[:N].set(x).T
    npe = EPAD - E
    padv = (N + (jnp.arange(npe, dtype=jnp.int32) % (NPAD - N))).astype(jnp.int32)
    srcp = jnp.concatenate([edge_index[0], padv])
    dstp = jnp.concatenate([edge_index[1], padv])
    zd = jnp.zeros((NPAD,), _F32)

    d0, d1 = _deg_call(dstp, zd)
    h1t = _tc_mm1t(W1.T, xpt)
    dis2d, g0, g1, g2, g3 = _tc_disg1(d0, d1, [h1t[f] for f in range(4)])
    a1 = _mp4_call(srcp, dstp, g0.reshape(NPAD), g1.reshape(NPAD),
                   g2.reshape(NPAD), g3.reshape(NPAD), zd)
    g20, g21 = _tc_mid(dis2d, a1, b1, W2.reshape(8))
    a2 = _mp2_call(srcp, dstp, g20.reshape(NPAD), g21.reshape(NPAD), zd)
    o0, o1 = _tc_fin(dis2d, a2, b2)
    return jnp.stack([o0.reshape(NPAD)[:N], o1.reshape(NPAD)[:N]], axis=1)


# speedup vs baseline: 121.7437x; 1.0356x over previous
import functools

import jax
import jax.numpy as jnp
from jax import lax
from jax.experimental import pallas as pl
from jax.experimental.pallas import tpu as pltpu
from jax.experimental.pallas import tpu_sc as plsc

N = 100000
E = 1600000
NC = 2
NS = 16
NW = NC * NS
NPAD = 100352
RPT = NPAD // NS
C = 25600
EPAD = 1638400
EPT = EPAD // NW
NCHUNK = EPT // C

_MESH = plsc.VectorSubcoreMesh(
    core_axis_name="c", subcore_axis_name="s", num_cores=NC, num_subcores=NS
)

_F32 = jnp.float32


def _nslice(ref, s):
    return ref.at[pl.ds(s * RPT, RPT)]


@functools.partial(
    pl.kernel,
    out_type=[
        jax.ShapeDtypeStruct((NPAD,), _F32),
        jax.ShapeDtypeStruct((NPAD,), _F32),
    ],
    mesh=_MESH,
    scratch_types=[
        pltpu.VMEM((C,), jnp.int32),
        pltpu.VMEM((C,), _F32),
        pltpu.VMEM_SHARED((NPAD,), _F32),
    ],
)
def _deg_call(dst_hbm, zeros_hbm, out0_hbm, out1_hbm, didx, ones_v, deg_sh):
    c = lax.axis_index("c")
    s = lax.axis_index("s")
    wid = c * NS + s
    for i in range(C // 16):
        ones_v[pl.ds(i * 16, 16)] = jnp.ones((16,), _F32)
    pltpu.sync_copy(_nslice(zeros_hbm, s), _nslice(deg_sh, s))
    plsc.subcore_barrier()

    ebase = wid * EPT

    def chunk(j, carry):
        pltpu.sync_copy(dst_hbm.at[pl.ds(ebase + j * C, C)], didx)
        pltpu.sync_copy(ones_v, deg_sh.at[didx], add=True)
        return carry

    lax.fori_loop(0, NCHUNK, chunk, 0)
    plsc.subcore_barrier()

    @pl.when(c == 0)
    def _():
        pltpu.sync_copy(_nslice(deg_sh, s), _nslice(out0_hbm, s))

    @pl.when(c != 0)
    def _():
        pltpu.sync_copy(_nslice(deg_sh, s), _nslice(out1_hbm, s))


def _mp_body(F, *refs):
    src_hbm, dst_hbm = refs[0], refs[1]
    g_hbm = refs[2:2 + F]
    z_hbm = refs[2 + F]
    outs = refs[3 + F:3 + 3 * F]
    sidx, didx, m_v = refs[3 + 3 * F:6 + 3 * F]
    g_sh = refs[6 + 3 * F:6 + 4 * F]
    a_sh = refs[6 + 4 * F:6 + 5 * F]

    c = lax.axis_index("c")
    s = lax.axis_index("s")
    wid = c * NS + s

    for f in range(F):
        pltpu.sync_copy(_nslice(g_hbm[f], s), _nslice(g_sh[f], s))

    @pl.when(c == 0)
    def _():
        for f in range(F):
            pltpu.sync_copy(_nslice(g_hbm[f], s), _nslice(a_sh[f], s))

    @pl.when(c != 0)
    def _():
        for f in range(F):
            pltpu.sync_copy(_nslice(z_hbm, s), _nslice(a_sh[f], s))

    plsc.subcore_barrier()

    ebase = wid * EPT

    def chunk(j, carry):
        pltpu.sync_copy(src_hbm.at[pl.ds(ebase + j * C, C)], sidx)
        pltpu.sync_copy(dst_hbm.at[pl.ds(ebase + j * C, C)], didx)
        for f in range(F):
            pltpu.sync_copy(g_sh[f].at[sidx], m_v)
            pltpu.sync_copy(m_v, a_sh[f].at[didx], add=True)
        return carry

    lax.fori_loop(0, NCHUNK, chunk, 0)
    plsc.subcore_barrier()

    @pl.when(c == 0)
    def _():
        for f in range(F):
            pltpu.sync_copy(_nslice(a_sh[f], s), _nslice(outs[f], s))

    @pl.when(c != 0)
    def _():
        for f in range(F):
            pltpu.sync_copy(_nslice(a_sh[f], s), _nslice(outs[F + f], s))


def _make_mp(F):
    return functools.partial(
        pl.kernel,
        out_type=[jax.ShapeDtypeStruct((NPAD,), _F32)] * (2 * F),
        mesh=_MESH,
        scratch_types=(
            [
                pltpu.VMEM((C,), jnp.int32),
                pltpu.VMEM((C,), jnp.int32),
                pltpu.VMEM((C,), _F32),
            ]
            + [pltpu.VMEM_SHARED((NPAD,), _F32)] * (2 * F)
        ),
    )(functools.partial(_mp_body, F))


_mp4_call = _make_mp(4)
_mp2_call = _make_mp(2)


_V2D = (NPAD // 128, 128)


def _vspec():
    return pl.BlockSpec(_V2D, lambda: (0, 0))


def _sspec():
    return pl.BlockSpec(memory_space=pltpu.SMEM)


def _mm1t_body(w1t_ref, xt_ref, h_ref):
    h_ref[...] = jnp.dot(w1t_ref[...], xt_ref[...], preferred_element_type=_F32)


def _tc_mm1t(w1t, xpt):
    return pl.pallas_call(
        _mm1t_body,
        grid=(NS,),
        in_specs=[
            pl.BlockSpec((4, 34), lambda i: (0, 0)),
            pl.BlockSpec((34, RPT), lambda i: (0, i)),
        ],
        out_specs=pl.BlockSpec((4, RPT), lambda i: (0, i)),
        out_shape=jax.ShapeDtypeStruct((4, NPAD), _F32),
    )(w1t, xpt)


def _disg1_body(d0_ref, d1_ref, h0_ref, h1_ref, h2_ref, h3_ref,
                dis_ref, g0_ref, g1_ref, g2_ref, g3_ref):
    dis = lax.rsqrt(d0_ref[...] + d1_ref[...] + 1.0)
    dis_ref[...] = dis
    g0_ref[...] = h0_ref[...] * dis
    g1_ref[...] = h1_ref[...] * dis
    g2_ref[...] = h2_ref[...] * dis
    g3_ref[...] = h3_ref[...] * dis


def _tc_disg1(d0, d1, h):
    return pl.pallas_call(
        _disg1_body,
        in_specs=[_vspec()] * 6,
        out_specs=[_vspec()] * 5,
        out_shape=[jax.ShapeDtypeStruct(_V2D, _F32)] * 5,
    )(d0.reshape(_V2D), d1.reshape(_V2D),
      h[0].reshape(_V2D), h[1].reshape(_V2D),
      h[2].reshape(_V2D), h[3].reshape(_V2D))


def _mid_body(dis_ref, a00, a01, a02, a03, a10, a11, a12, a13,
              b1_ref, w2_ref, g0_ref, g1_ref):
    dis = dis_ref[...]
    o = [
        jnp.maximum(dis * (a0[...] + a1[...]) + b1_ref[f], 0.0)
        for f, (a0, a1) in enumerate(
            [(a00, a10), (a01, a11), (a02, a12), (a03, a13)]
        )
    ]
    for j, g_ref in enumerate((g0_ref, g1_ref)):
        h = o[0] * w2_ref[0 * 2 + j]
        for f in range(1, 4):
            h = h + o[f] * w2_ref[f * 2 + j]
        g_ref[...] = h * dis


def _tc_mid(dis, a, b1, w2f):
    return pl.pallas_call(
        _mid_body,
        in_specs=[_vspec()] * 9 + [_sspec(), _sspec()],
        out_specs=[_vspec()] * 2,
        out_shape=[jax.ShapeDtypeStruct(_V2D, _F32)] * 2,
    )(dis, *[x.reshape(_V2D) for x in a], b1, w2f)


def _fin_body(dis_ref, a00, a01, a10, a11, b2_ref, o0_ref, o1_ref):
    dis = dis_ref[...]
    z0 = dis * (a00[...] + a10[...]) + b2_ref[0]
    z1 = dis * (a01[...] + a11[...]) + b2_ref[1]
    m = jnp.maximum(z0, z1)
    e0 = jnp.exp(z0 - m)
    e1 = jnp.exp(z1 - m)
    inv = 1.0 / (e0 + e1)
    o0_ref[...] = e0 * inv
    o1_ref[...] = e1 * inv


def _tc_fin(dis, a, b2):
    return pl.pallas_call(
        _fin_body,
        in_specs=[_vspec()] * 5 + [_sspec()],
        out_specs=[_vspec()] * 2,
        out_shape=[jax.ShapeDtypeStruct(_V2D, _F32)] * 2,
    )(dis, *[x.reshape(_V2D) for x in a], b2)


def kernel(x, edge_index, W1, b1, W2, b2):
    xpt = jnp.zeros((NPAD, 34), _F32).at[:N].set(x).T
    npe = EPAD - E
    padv = (N + (jnp.arange(npe, dtype=jnp.int32) % (NPAD - N))).astype(jnp.int32)
    srcp = jnp.concatenate([edge_index[0], padv])
    dstp = jnp.concatenate([edge_index[1], padv])
    zd = jnp.zeros((NPAD,), _F32)

    d0, d1 = _deg_call(dstp, zd)
    h1t = _tc_mm1t(W1.T, xpt)
    dis2d, g0, g1, g2, g3 = _tc_disg1(d0, d1, [h1t[f] for f in range(4)])
    a1 = _mp4_call(srcp, dstp, g0.reshape(NPAD), g1.reshape(NPAD),
                   g2.reshape(NPAD), g3.reshape(NPAD), zd)
    g20, g21 = _tc_mid(dis2d, a1, b1, W2.reshape(8))
    a2 = _mp2_call(srcp, dstp, g20.reshape(NPAD), g21.reshape(NPAD), zd)
    o0, o1 = _tc_fin(dis2d, a2, b2)
    return jnp.stack([o0.reshape(NPAD)[:N], o1.reshape(NPAD)[:N]], axis=1)


# speedup vs baseline: 125.7331x; 1.0328x over previous
import functools

import jax
import jax.numpy as jnp
from jax import lax
from jax.experimental import pallas as pl
from jax.experimental.pallas import tpu as pltpu
from jax.experimental.pallas import tpu_sc as plsc

N = 100000
E = 1600000
NC = 2
NS = 16
NW = NC * NS
NPAD = 100352
RPT = NPAD // NS
EPT = E // NW
CD = 50000
C4 = 10000
C2 = 25000

_MESH = plsc.VectorSubcoreMesh(
    core_axis_name="c", subcore_axis_name="s", num_cores=NC, num_subcores=NS
)

_F32 = jnp.float32


def _nslice(ref, s):
    return ref.at[pl.ds(s * RPT, RPT)]


@functools.partial(
    pl.kernel,
    out_type=[
        jax.ShapeDtypeStruct((NPAD,), _F32),
        jax.ShapeDtypeStruct((NPAD,), _F32),
    ],
    mesh=_MESH,
    scratch_types=[
        pltpu.VMEM((CD,), jnp.int32),
        pltpu.VMEM((CD,), _F32),
        pltpu.VMEM_SHARED((NPAD,), _F32),
    ],
)
def _deg_call(dst_hbm, zeros_hbm, out0_hbm, out1_hbm, didx, ones_v, deg_sh):
    c = lax.axis_index("c")
    s = lax.axis_index("s")
    wid = c * NS + s

    def fill(i, carry):
        ones_v[pl.ds(i * 16, 16)] = jnp.ones((16,), _F32)
        return carry

    lax.fori_loop(0, CD // 16, fill, 0)
    pltpu.sync_copy(_nslice(zeros_hbm, s), _nslice(deg_sh, s))
    plsc.subcore_barrier()

    ebase = wid * EPT

    def chunk(j, carry):
        pltpu.sync_copy(dst_hbm.at[pl.ds(ebase + j * CD, CD)], didx)
        pltpu.sync_copy(ones_v, deg_sh.at[didx], add=True)
        return carry

    lax.fori_loop(0, EPT // CD, chunk, 0)
    plsc.subcore_barrier()

    @pl.when(c == 0)
    def _():
        pltpu.sync_copy(_nslice(deg_sh, s), _nslice(out0_hbm, s))

    @pl.when(c != 0)
    def _():
        pltpu.sync_copy(_nslice(deg_sh, s), _nslice(out1_hbm, s))


def _mp_body(F, C, *refs):
    src_hbm, dst_hbm = refs[0], refs[1]
    g_hbm = refs[2:2 + F]
    z_hbm = refs[2 + F]
    outs = refs[3 + F:3 + 3 * F]
    k = 3 + 3 * F
    sidx, didx = refs[k], refs[k + 1]
    m_v = refs[k + 2:k + 2 + F]
    g_sh = refs[k + 2 + F:k + 2 + 2 * F]
    a_sh = refs[k + 2 + 2 * F:k + 2 + 3 * F]
    sem_i, sem_g, sem_s = refs[k + 2 + 3 * F:k + 5 + 3 * F]

    c = lax.axis_index("c")
    s = lax.axis_index("s")
    wid = c * NS + s

    stage = [
        pltpu.async_copy(_nslice(g_hbm[f], s), _nslice(g_sh[f], s), sem_g)
        for f in range(F)
    ]

    @pl.when(c == 0)
    def _():
        for f in range(F):
            pltpu.async_copy(_nslice(g_hbm[f], s), _nslice(a_sh[f], s), sem_s).wait()

    @pl.when(c != 0)
    def _():
        for f in range(F):
            pltpu.async_copy(_nslice(z_hbm, s), _nslice(a_sh[f], s), sem_s).wait()

    for d in stage:
        d.wait()
    plsc.subcore_barrier()

    ebase = wid * EPT

    def chunk(j, carry):
        base = ebase + j * C
        di0 = pltpu.async_copy(src_hbm.at[pl.ds(base, C)], sidx, sem_i)
        di1 = pltpu.async_copy(dst_hbm.at[pl.ds(base, C)], didx, sem_i)
        di0.wait()
        di1.wait()
        gs = [
            pltpu.async_copy(g_sh[f].at[sidx], m_v[f], sem_g)
            for f in range(F)
        ]
        for d in gs:
            d.wait()
        ss = [
            pltpu.async_copy(m_v[f], a_sh[f].at[didx], sem_s, add=True)
            for f in range(F)
        ]
        for d in ss:
            d.wait()
        return carry

    lax.fori_loop(0, EPT // C, chunk, 0)
    plsc.subcore_barrier()

    @pl.when(c == 0)
    def _():
        for f in range(F):
            pltpu.async_copy(_nslice(a_sh[f], s), _nslice(outs[f], s), sem_g).wait()

    @pl.when(c != 0)
    def _():
        for f in range(F):
            pltpu.async_copy(_nslice(a_sh[f], s), _nslice(outs[F + f], s), sem_g).wait()


def _make_mp(F, C):
    return functools.partial(
        pl.kernel,
        out_type=[jax.ShapeDtypeStruct((NPAD,), _F32)] * (2 * F),
        mesh=_MESH,
        scratch_types=(
            [
                pltpu.VMEM((C,), jnp.int32),
                pltpu.VMEM((C,), jnp.int32),
            ]
            + [pltpu.VMEM((C,), _F32)] * F
            + [pltpu.VMEM_SHARED((NPAD,), _F32)] * (2 * F)
            + [pltpu.SemaphoreType.DMA] * 3
        ),
    )(functools.partial(_mp_body, F, C))


_mp4_call = _make_mp(4, C4)
_mp2_call = _make_mp(2, C2)


_V2D = (NPAD // 128, 128)


def _vspec():
    return pl.BlockSpec(_V2D, lambda: (0, 0))


def _sspec():
    return pl.BlockSpec(memory_space=pltpu.SMEM)


def _mm1t_body(w1t_ref, xt_ref, h_ref):
    h_ref[...] = jnp.dot(w1t_ref[...], xt_ref[...], preferred_element_type=_F32)


def _tc_mm1t(w1t, xpt):
    return pl.pallas_call(
        _mm1t_body,
        grid=(NS,),
        in_specs=[
            pl.BlockSpec((4, 34), lambda i: (0, 0)),
            pl.BlockSpec((34, RPT), lambda i: (0, i)),
        ],
        out_specs=pl.BlockSpec((4, RPT), lambda i: (0, i)),
        out_shape=jax.ShapeDtypeStruct((4, NPAD), _F32),
    )(w1t, xpt)


def _disg1_body(d0_ref, d1_ref, h0_ref, h1_ref, h2_ref, h3_ref,
                dis_ref, g0_ref, g1_ref, g2_ref, g3_ref):
    dis = lax.rsqrt(d0_ref[...] + d1_ref[...] + 1.0)
    dis_ref[...] = dis
    g0_ref[...] = h0_ref[...] * dis
    g1_ref[...] = h1_ref[...] * dis
    g2_ref[...] = h2_ref[...] * dis
    g3_ref[...] = h3_ref[...] * dis


def _tc_disg1(d0, d1, h):
    return pl.pallas_call(
        _disg1_body,
        in_specs=[_vspec()] * 6,
        out_specs=[_vspec()] * 5,
        out_shape=[jax.ShapeDtypeStruct(_V2D, _F32)] * 5,
    )(d0.reshape(_V2D), d1.reshape(_V2D),
      h[0].reshape(_V2D), h[1].reshape(_V2D),
      h[2].reshape(_V2D), h[3].reshape(_V2D))


def _mid_body(dis_ref, a00, a01, a02, a03, a10, a11, a12, a13,
              b1_ref, w2_ref, g0_ref, g1_ref):
    dis = dis_ref[...]
    o = [
        jnp.maximum(dis * (a0[...] + a1[...]) + b1_ref[f], 0.0)
        for f, (a0, a1) in enumerate(
            [(a00, a10), (a01, a11), (a02, a12), (a03, a13)]
        )
    ]
    for j, g_ref in enumerate((g0_ref, g1_ref)):
        h = o[0] * w2_ref[0 * 2 + j]
        for f in range(1, 4):
            h = h + o[f] * w2_ref[f * 2 + j]
        g_ref[...] = h * dis


def _tc_mid(dis, a, b1, w2f):
    return pl.pallas_call(
        _mid_body,
        in_specs=[_vspec()] * 9 + [_sspec(), _sspec()],
        out_specs=[_vspec()] * 2,
        out_shape=[jax.ShapeDtypeStruct(_V2D, _F32)] * 2,
    )(dis, *[x.reshape(_V2D) for x in a], b1, w2f)


def _fin_body(dis_ref, a00, a01, a10, a11, b2_ref, o0_ref, o1_ref):
    dis = dis_ref[...]
    z0 = dis * (a00[...] + a10[...]) + b2_ref[0]
    z1 = dis * (a01[...] + a11[...]) + b2_ref[1]
    m = jnp.maximum(z0, z1)
    e0 = jnp.exp(z0 - m)
    e1 = jnp.exp(z1 - m)
    inv = 1.0 / (e0 + e1)
    o0_ref[...] = e0 * inv
    o1_ref[...] = e1 * inv


def _tc_fin(dis, a, b2):
    return pl.pallas_call(
        _fin_body,
        in_specs=[_vspec()] * 5 + [_sspec()],
        out_specs=[_vspec()] * 2,
        out_shape=[jax.ShapeDtypeStruct(_V2D, _F32)] * 2,
    )(dis, *[x.reshape(_V2D) for x in a], b2)


def kernel(x, edge_index, W1, b1, W2, b2):
    xpt = jnp.zeros((NPAD, 34), _F32).at[:N].set(x).T
    srcp = edge_index[0]
    dstp = edge_index[1]
    zd = jnp.zeros((NPAD,), _F32)

    d0, d1 = _deg_call(dstp, zd)
    h1t = _tc_mm1t(W1.T, xpt)
    dis2d, g0, g1, g2, g3 = _tc_disg1(d0, d1, [h1t[f] for f in range(4)])
    a1 = _mp4_call(srcp, dstp, g0.reshape(NPAD), g1.reshape(NPAD),
                   g2.reshape(NPAD), g3.reshape(NPAD), zd)
    g20, g21 = _tc_mid(dis2d, a1, b1, W2.reshape(8))
    a2 = _mp2_call(srcp, dstp, g20.reshape(NPAD), g21.reshape(NPAD), zd)
    o0, o1 = _tc_fin(dis2d, a2, b2)
    return jnp.stack([o0.reshape(NPAD)[:N], o1.reshape(NPAD)[:N]], axis=1)
